# baseline (speedup 1.0000x reference)
import functools

import jax
import jax.numpy as jnp
from jax import lax
from jax.experimental import pallas as pl
from jax.experimental.pallas import tpu as pltpu
from jax.experimental.pallas import tpu_sc as plsc

_TILE = 8


def _expand_sc(table, batch_size):
    info = plsc.get_sparse_core_info()
    nw = info.num_cores * info.num_subcores
    nqueries, d = table.shape
    ngroups = batch_size // _TILE
    nq = -(-nqueries // nw)
    lbuf = 3 * _TILE
    pad_rows = -(-nqueries // _TILE) * _TILE

    mesh = plsc.VectorSubcoreMesh(core_axis_name="c", subcore_axis_name="s")

    @functools.partial(
        pl.kernel,
        mesh=mesh,
        out_type=jax.ShapeDtypeStruct(
            (nqueries, ngroups, _TILE, d), jnp.float32
        ),
        scratch_types=[
            pltpu.VMEM((lbuf, d), jnp.float32),
            pltpu.VMEM((nq, _TILE, d), jnp.float32),
            pltpu.SemaphoreType.DMA,
        ],
    )
    def k(table_hbm, out_hbm, tab_v, rep_v, sem):
        wid = lax.axis_index("s") * info.num_cores + lax.axis_index("c")
        q0 = (wid * nqueries) // nw
        q0a = lax.min((q0 // _TILE) * _TILE, pad_rows - lbuf)
        off = q0 - q0a
        pltpu.sync_copy(table_hbm.at[pl.ds(q0a, lbuf)], tab_v)
        chunks = []
        pos = 0
        for cn in (1, 2, 3, 4):
            chunks.append((pos, cn))
            pos += cn
        assert pos == nq
        copies = []
        for h, hn in chunks:
            for j in range(h, h + hn):
                for i in range(d // 16):
                    v = tab_v[off + j, pl.ds(i * 16, 16)]
                    for t in range(_TILE):
                        rep_v[j, t, pl.ds(i * 16, 16)] = v
            copies += [
                pltpu.async_copy(
                    rep_v.at[pl.ds(h, hn)],
                    out_hbm.at[pl.ds(q0 + h, hn), g],
                    sem,
                )
                for g in range(ngroups)
            ]
        for c in copies:
            c.wait()

    return k(table)


def kernel(batch_ref, table):
    batch_size = batch_ref.shape[0]
    out4 = _expand_sc(table, batch_size)
    out3 = out4.reshape(table.shape[0], batch_size, table.shape[1])
    return jnp.transpose(out3, (1, 0, 2))
